# baseline (speedup 1.0000x reference)
import functools

import jax
import jax.numpy as jnp
from jax import lax
from jax.experimental import pallas as pl
from jax.experimental.pallas import tpu as pltpu
from jax.experimental.pallas import tpu_sc as plsc

D = 128
B = 16384
NR = 1000
NC, NS, L = 2, 16, 16
NW = NC * NS
RPW = B // NW
CH = 64
NCHUNK = RPW // CH
NBUF = 3
PRE = 64


def _rsqrt_nr(x):
    xi = plsc.bitcast(x, jnp.int32)
    yi = jnp.int32(0x5F3759DF) - (xi >> 1)
    y = plsc.bitcast(yi, jnp.float32)
    for _ in range(3):
        y = y * (1.5 - (0.5 * x) * y * y)
    return y


def _make_kernel():
    mesh = plsc.VectorSubcoreMesh(core_axis_name="c", subcore_axis_name="s")

    @functools.partial(
        pl.kernel,
        mesh=mesh,
        out_type=jax.ShapeDtypeStruct((B,), jnp.float32),
        compiler_params=pltpu.CompilerParams(needs_layout_passes=False),
        scratch_types=[
            pltpu.VMEM((RPW,), jnp.int32),
            pltpu.VMEM((RPW,), jnp.int32),
            pltpu.VMEM((RPW,), jnp.int32),
            pltpu.VMEM((NBUF, CH, D), jnp.float32),
            pltpu.VMEM((NBUF, CH, D), jnp.float32),
            pltpu.VMEM((NBUF, CH, D), jnp.float32),
            pltpu.VMEM((PRE, D), jnp.float32),
            pltpu.VMEM((PRE, D), jnp.float32),
            pltpu.VMEM((PRE,), jnp.float32),
            pltpu.VMEM((NR,), jnp.float32),
            pltpu.VMEM_SHARED((NR,), jnp.float32),
            pltpu.VMEM((RPW,), jnp.float32),
            pltpu.VMEM((RPW,), jnp.float32),
            pltpu.VMEM((RPW,), jnp.float32),
            pltpu.VMEM((RPW,), jnp.float32),
            pltpu.SemaphoreType.DMA,
            pltpu.SemaphoreType.DMA,
            pltpu.SemaphoreType.DMA,
            pltpu.SemaphoreType.DMA,
            pltpu.SemaphoreType.DMA,
            pltpu.SemaphoreType.DMA,
            pltpu.SemaphoreType.DMA,
        ],
    )
    def vec2tail(h_hbm, r_hbm, t_hbm, ent_hbm, hyp_hbm, nrm_hbm, out_hbm,
                 hidx, tidx, ridx, hbuf, tbuf, nbuf, pre_d, pre_n, beta_loc,
                 beta_vm, beta_sh, qbuf, pbuf, vbuf, obuf,
                 isem, hsem0, hsem1, hsem2, rsem0, rsem1, rsem2):
        wid = lax.axis_index("s") * NC + lax.axis_index("c")
        tid = lax.axis_index("s")
        base = wid * RPW
        icps = [
            pltpu.async_copy(h_hbm.at[pl.ds(base, RPW)], hidx, isem),
            pltpu.async_copy(t_hbm.at[pl.ds(base, RPW)], tidx, isem),
            pltpu.async_copy(r_hbm.at[pl.ds(base, RPW)], ridx, isem),
        ]
        start = jnp.minimum(tid * PRE, NR - PRE)
        pcps = [
            pltpu.async_copy(hyp_hbm.at[pl.ds(start, PRE)], pre_d, isem),
            pltpu.async_copy(nrm_hbm.at[pl.ds(start, PRE)], pre_n, isem),
        ]
        for cp in icps:
            cp.wait()

        hsems = (hsem0, hsem1, hsem2)
        rsems = (rsem0, rsem1, rsem2)
        lane15 = lax.iota(jnp.int32, L) == (L - 1)

        def fire_htn(g):
            slot = g % NBUF
            sl = pl.ds(g * CH, CH)
            return (
                pltpu.async_copy(ent_hbm.at[hidx.at[sl]], hbuf.at[slot],
                                 hsems[slot]),
                pltpu.async_copy(ent_hbm.at[tidx.at[sl]], tbuf.at[slot],
                                 rsems[slot]),
                pltpu.async_copy(nrm_hbm.at[ridx.at[sl]], nbuf.at[slot],
                                 rsems[slot]),
            )

        def fire_add(g, htn):
            htn[0].wait()
            slot = g % NBUF
            sl = pl.ds(g * CH, CH)
            return pltpu.async_copy(hyp_hbm.at[ridx.at[sl]], hbuf.at[slot],
                                    rsems[slot], add=True)

        pending = {g: fire_htn(g) for g in range(min(2, NCHUNK))}

        for cp in pcps:
            cp.wait()

        @plsc.parallel_loop(0, PRE, 1, unroll=4)
        def beta_row(j):
            accb = None
            for cb in range(D // L):
                sl = pl.ds(cb * L, L)
                prod = pre_d[j, sl] * pre_n[j, sl]
                accb = prod if cb == 0 else accb + prod
            bb = plsc.cumsum(accb)
            plsc.store_scatter(
                beta_loc, [jnp.broadcast_to(j, (L,))], bb, mask=lane15)

        pltpu.sync_copy(beta_loc, beta_sh.at[pl.ds(start, PRE)])
        plsc.subcore_barrier()
        pltpu.sync_copy(beta_sh, beta_vm)

        adds = {0: fire_add(0, pending[0])}
        for g in range(NCHUNK):
            if g + 1 < NCHUNK:
                adds[g + 1] = fire_add(g + 1, pending[g + 1])
            if g + 2 < NCHUNK:
                pending[g + 2] = fire_htn(g + 2)
            cps = pending.pop(g)
            cps[1].wait()
            cps[2].wait()
            adds.pop(g).wait()
            slot = g % NBUF

            @plsc.parallel_loop(0, CH, 1, unroll=4)
            def row(i, slot=slot, g=g):
                accq = accp = accn = None
                for cb in range(D // L):
                    sl = pl.ds(cb * L, L)
                    uv = hbuf[slot, i, sl]
                    tv = tbuf[slot, i, sl]
                    nv = nbuf[slot, i, sl]
                    u = uv - tv
                    if cb == 0:
                        accq, accp, accn = u * u, u * nv, nv * nv
                    else:
                        accq = accq + u * u
                        accp = accp + u * nv
                        accn = accn + nv * nv
                idx = jnp.broadcast_to(g * CH + i, (L,))
                plsc.store_scatter(qbuf, [idx], plsc.cumsum(accq), mask=lane15)
                plsc.store_scatter(pbuf, [idx], plsc.cumsum(accp), mask=lane15)
                plsc.store_scatter(vbuf, [idx], plsc.cumsum(accn), mask=lane15)

        for k in range(RPW // L):
            sl = pl.ds(k * L, L)
            rv = ridx[sl]
            bv = plsc.load_gather(beta_vm, [rv])
            q2 = qbuf[sl]
            pn = pbuf[sl]
            n2 = vbuf[sl]
            c = (pn - bv) / jnp.maximum(n2, 1e-24)
            s = jnp.maximum(q2 - (2.0 * c) * pn + (c * c) * n2, 0.0)
            obuf[sl] = s * _rsqrt_nr(s)
        pltpu.sync_copy(obuf, out_hbm.at[pl.ds(base, RPW)])

    return vec2tail


_vec2tail = _make_kernel()


def kernel(h, r, t, ent_embedding, rel_hyperplane_embedding,
           rel_norm_embedding):
    return _vec2tail(h.astype(jnp.int32), r.astype(jnp.int32),
                     t.astype(jnp.int32), ent_embedding,
                     rel_hyperplane_embedding, rel_norm_embedding)
